# Initial kernel scaffold; baseline (speedup 1.0000x reference)
import functools

import jax
import jax.numpy as jnp
from jax import lax
from jax.experimental import pallas as pl
from jax.experimental.pallas import tpu as pltpu
from jax.experimental.pallas import tpu_sc as plsc

_GAMMA = 1.0
_ALPHA = 0.0001
_BATCH = 4096
_D = 128
_NW = 32
_BPW = _BATCH // _NW


_sc_mesh = plsc.VectorSubcoreMesh(core_axis_name="c", subcore_axis_name="s")


@functools.partial(
    pl.kernel,
    out_type=jax.ShapeDtypeStruct((9 * _BATCH, _D), jnp.float32),
    mesh=_sc_mesh,
    scratch_types=[
        pltpu.VMEM((_BPW,), jnp.int32),
        pltpu.VMEM((_BPW, _D), jnp.float32),
        pltpu.SemaphoreType.DMA,
    ],
)
def _sc_gather(ent_hbm, rel_hbm, idx_hbm, out_hbm, idx_v, rows_v, sem):
    wid = lax.axis_index("s") * 2 + lax.axis_index("c")
    base = wid * _BPW
    for a in range(9):
        table = rel_hbm if a in (2, 5) else ent_hbm
        off = a * _BATCH + base
        pltpu.sync_copy(idx_hbm.at[pl.ds(off, _BPW)], idx_v)
        pltpu.async_copy(table.at[idx_v], rows_v, sem).wait()
        pltpu.sync_copy(rows_v, out_hbm.at[pl.ds(off, _BPW)])


def _score_body(g_ref, o_ref):
    def rows(a):
        return g_ref[a * _BATCH:(a + 1) * _BATCH, :]

    def norm_rows(x):
        return x * lax.rsqrt(jnp.sum(x * x, axis=1, keepdims=True))

    h = norm_rows(rows(0))
    t = norm_rows(rows(1))
    r = norm_rows(rows(2))
    nh = norm_rows(rows(3))
    nt = norm_rows(rows(4))
    nr = norm_rows(rows(5))
    d1 = h + r - t
    d2 = nh + nr - nt
    s = (_GAMMA
         + jnp.sqrt(jnp.sum(d1 * d1, axis=1))
         - jnp.sqrt(jnp.sum(d2 * d2, axis=1)))
    reg = jnp.float32(0.0)
    for a in (6, 7, 8):
        a_rows = rows(a)
        gram = lax.dot_general(a_rows, a_rows, (((0,), (0,)), ((), ())),
                               preferred_element_type=jnp.float32)
        reg = reg + jnp.sqrt(jnp.sum(gram * gram))
    o_ref[...] = s + _ALPHA * reg


_score_call = pl.pallas_call(
    _score_body,
    out_shape=jax.ShapeDtypeStruct((_BATCH,), jnp.float32),
)


def kernel(head, tail, relation, n_head, n_tail, n_relation, reg_user,
           reg_item, reg_brand, entity_embed, relation_embed):
    idx_all = jnp.concatenate([
        head, tail, relation, n_head, n_tail, n_relation,
        reg_user, reg_item, reg_brand,
    ]).astype(jnp.int32)
    gathered = _sc_gather(entity_embed, relation_embed, idx_all)
    return _score_call(gathered)


# speedup vs baseline: 3.3994x; 3.3994x over previous
import functools

import jax
import jax.numpy as jnp
from jax import lax
from jax.experimental import pallas as pl
from jax.experimental.pallas import tpu as pltpu
from jax.experimental.pallas import tpu_sc as plsc

_GAMMA = 1.0
_ALPHA = 0.0001
_BATCH = 4096
_D = 128
_NW = 32
_BPW = _BATCH // _NW


@functools.cache
def _sc_gather_call():
    mesh = plsc.VectorSubcoreMesh(core_axis_name="c", subcore_axis_name="s")

    @functools.partial(
        pl.kernel,
        out_type=jax.ShapeDtypeStruct((9 * _BATCH, _D), jnp.float32),
        mesh=mesh,
        scratch_types=[
            pltpu.VMEM((_BPW,), jnp.int32),
            pltpu.VMEM((_BPW, _D), jnp.float32),
            pltpu.SemaphoreType.DMA,
        ],
    )
    def _sc_gather(ent_hbm, rel_hbm, idx_hbm, out_hbm, idx_v, rows_v, sem):
        wid = lax.axis_index("s") * 2 + lax.axis_index("c")
        base = wid * _BPW
        for a in range(9):
            table = rel_hbm if a in (2, 5) else ent_hbm
            off = a * _BATCH + base
            pltpu.sync_copy(idx_hbm.at[pl.ds(off, _BPW)], idx_v)
            pltpu.async_copy(table.at[idx_v], rows_v, sem).wait()
            pltpu.sync_copy(rows_v, out_hbm.at[pl.ds(off, _BPW)])

    return _sc_gather


def _score_body(g_ref, o_ref):
    def rows(a):
        return g_ref[a * _BATCH:(a + 1) * _BATCH, :]

    def norm_rows(x):
        return x * lax.rsqrt(jnp.sum(x * x, axis=1, keepdims=True))

    h = norm_rows(rows(0))
    t = norm_rows(rows(1))
    r = norm_rows(rows(2))
    nh = norm_rows(rows(3))
    nt = norm_rows(rows(4))
    nr = norm_rows(rows(5))
    d1 = h + r - t
    d2 = nh + nr - nt
    s = (_GAMMA
         + jnp.sqrt(jnp.sum(d1 * d1, axis=1))
         - jnp.sqrt(jnp.sum(d2 * d2, axis=1)))
    reg = jnp.float32(0.0)
    for a in (6, 7, 8):
        a_rows = rows(a)
        gram = lax.dot_general(a_rows, a_rows, (((0,), (0,)), ((), ())),
                               preferred_element_type=jnp.float32)
        reg = reg + jnp.sqrt(jnp.sum(gram * gram))
    o_ref[...] = s + _ALPHA * reg


_score_call = pl.pallas_call(
    _score_body,
    out_shape=jax.ShapeDtypeStruct((_BATCH,), jnp.float32),
)


def kernel(head, tail, relation, n_head, n_tail, n_relation, reg_user,
           reg_item, reg_brand, entity_embed, relation_embed):
    idx_all = jnp.concatenate([
        head, tail, relation, n_head, n_tail, n_relation,
        reg_user, reg_item, reg_brand,
    ]).astype(jnp.int32)
    gathered = _sc_gather_call()(entity_embed, relation_embed, idx_all)
    return _score_call(gathered)


# speedup vs baseline: 4.4103x; 1.2974x over previous
import functools

import jax
import jax.numpy as jnp
from jax import lax
from jax.experimental import pallas as pl
from jax.experimental.pallas import tpu as pltpu
from jax.experimental.pallas import tpu_sc as plsc

_GAMMA = 1.0
_ALPHA = 0.0001
_BATCH = 4096
_D = 128
_NW = 32
_BPW = _BATCH // _NW


_NBUF = 4


@functools.cache
def _sc_gather_call():
    mesh = plsc.VectorSubcoreMesh(core_axis_name="c", subcore_axis_name="s")

    @functools.partial(
        pl.kernel,
        out_type=jax.ShapeDtypeStruct((9 * _BATCH, _D), jnp.float32),
        mesh=mesh,
        scratch_types=(
            [pltpu.VMEM((9, _BPW), jnp.int32),
             pltpu.VMEM((_NBUF, _BPW, _D), jnp.float32),
             pltpu.SemaphoreType.DMA]
            + [pltpu.SemaphoreType.DMA] * (2 * _NBUF)
        ),
    )
    def _sc_gather(i0, i1, i2, i3, i4, i5, i6, i7, i8, ent_hbm, rel_hbm,
                   out_hbm, idx_v, rows_v, isem, *bufsems):
        gsem = bufsems[:_NBUF]
        wsem = bufsems[_NBUF:]
        idx_refs = (i0, i1, i2, i3, i4, i5, i6, i7, i8)
        wid = lax.axis_index("s") * 2 + lax.axis_index("c")
        base = wid * _BPW

        icopies = [
            pltpu.async_copy(idx_refs[a].at[pl.ds(base, _BPW)], idx_v.at[a],
                             isem)
            for a in range(9)
        ]
        for c in icopies:
            c.wait()

        def tbl(a):
            return rel_hbm if a in (2, 5) else ent_hbm

        def gather(a):
            b = a % _NBUF
            return pltpu.async_copy(tbl(a).at[idx_v.at[a]], rows_v.at[b],
                                    gsem[b])

        g = {}
        w = {}
        g[0] = gather(0)
        g[1] = gather(1)
        for a in range(9):
            b = a % _NBUF
            g[a].wait()
            if a + 2 < 9:
                if a + 2 - _NBUF >= 0:
                    w[a + 2 - _NBUF].wait()
                g[a + 2] = gather(a + 2)
            w[a] = pltpu.async_copy(
                rows_v.at[b], out_hbm.at[pl.ds(a * _BATCH + base, _BPW)],
                wsem[b])
        for a in range(9 - _NBUF, 9):
            w[a].wait()

    return _sc_gather


def _score_body(g_ref, o_ref):
    def rows(a):
        return g_ref[a * _BATCH:(a + 1) * _BATCH, :]

    def norm_rows(x):
        return x * lax.rsqrt(jnp.sum(x * x, axis=1, keepdims=True))

    h = norm_rows(rows(0))
    t = norm_rows(rows(1))
    r = norm_rows(rows(2))
    nh = norm_rows(rows(3))
    nt = norm_rows(rows(4))
    nr = norm_rows(rows(5))
    d1 = h + r - t
    d2 = nh + nr - nt
    s = (_GAMMA
         + jnp.sqrt(jnp.sum(d1 * d1, axis=1))
         - jnp.sqrt(jnp.sum(d2 * d2, axis=1)))
    reg = jnp.float32(0.0)
    for a in (6, 7, 8):
        a_rows = rows(a)
        gram = lax.dot_general(a_rows, a_rows, (((0,), (0,)), ((), ())),
                               preferred_element_type=jnp.float32)
        reg = reg + jnp.sqrt(jnp.sum(gram * gram))
    o_ref[...] = s + _ALPHA * reg


_score_call = pl.pallas_call(
    _score_body,
    out_shape=jax.ShapeDtypeStruct((_BATCH,), jnp.float32),
)


def kernel(head, tail, relation, n_head, n_tail, n_relation, reg_user,
           reg_item, reg_brand, entity_embed, relation_embed):
    idxs = [x.astype(jnp.int32) for x in (
        head, tail, relation, n_head, n_tail, n_relation,
        reg_user, reg_item, reg_brand)]
    gathered = _sc_gather_call()(*idxs, entity_embed, relation_embed)
    return _score_call(gathered)


# speedup vs baseline: 4.4282x; 1.0041x over previous
import functools

import jax
import jax.numpy as jnp
from jax import lax
from jax.experimental import pallas as pl
from jax.experimental.pallas import tpu as pltpu
from jax.experimental.pallas import tpu_sc as plsc

_GAMMA = 1.0
_ALPHA = 0.0001
_BATCH = 4096
_D = 128
_NW = 32
_BPW = _BATCH // _NW


_NBUF = 4


@functools.cache
def _sc_gather_call():
    mesh = plsc.VectorSubcoreMesh(core_axis_name="c", subcore_axis_name="s")

    @functools.partial(
        pl.kernel,
        out_type=jax.ShapeDtypeStruct((9 * _BATCH, _D), jnp.float32),
        mesh=mesh,
        scratch_types=(
            [pltpu.VMEM((9, _BPW), jnp.int32),
             pltpu.VMEM((_NBUF, _BPW, _D), jnp.float32),
             pltpu.SemaphoreType.DMA]
            + [pltpu.SemaphoreType.DMA] * (2 * _NBUF)
        ),
    )
    def _sc_gather(i0, i1, i2, i3, i4, i5, i6, i7, i8, ent_hbm, rel_hbm,
                   out_hbm, idx_v, rows_v, isem, *bufsems):
        gsem = bufsems[:_NBUF]
        wsem = bufsems[_NBUF:]
        idx_refs = (i0, i1, i2, i3, i4, i5, i6, i7, i8)
        wid = lax.axis_index("s") * 2 + lax.axis_index("c")
        base = wid * _BPW

        icopies = [
            pltpu.async_copy(idx_refs[a].at[pl.ds(base, _BPW)], idx_v.at[a],
                             isem)
            for a in range(9)
        ]
        for c in icopies:
            c.wait()

        def tbl(a):
            return rel_hbm if a in (2, 5) else ent_hbm

        def gather(a):
            b = a % _NBUF
            return pltpu.async_copy(tbl(a).at[idx_v.at[a]], rows_v.at[b],
                                    gsem[b])

        g = {}
        w = {}
        g[0] = gather(0)
        g[1] = gather(1)
        for a in range(9):
            b = a % _NBUF
            g[a].wait()
            if a + 2 < 9:
                if a + 2 - _NBUF >= 0:
                    w[a + 2 - _NBUF].wait()
                g[a + 2] = gather(a + 2)
            w[a] = pltpu.async_copy(
                rows_v.at[b], out_hbm.at[pl.ds(a * _BATCH + base, _BPW)],
                wsem[b])
        for a in range(9 - _NBUF, 9):
            w[a].wait()

    return _sc_gather


def _score_body(g_ref, o_ref, s1_ref, s2_ref):
    i = pl.program_id(0)

    def rows(a):
        return g_ref[0, a * _BATCH:(a + 1) * _BATCH, :]

    def norm_rows(x):
        return x * lax.rsqrt(jnp.sum(x * x, axis=1, keepdims=True))

    def half_dist():
        d = norm_rows(rows(0)) + norm_rows(rows(2)) - norm_rows(rows(1))
        return jnp.sqrt(jnp.sum(d * d, axis=1))

    @pl.when(i == 0)
    def _():
        s1_ref[...] = half_dist()

    @pl.when(i == 1)
    def _():
        s2_ref[...] = half_dist()

    @pl.when(i == 2)
    def _():
        reg = jnp.float32(0.0)
        for a in range(3):
            a_rows = rows(a)
            gram = lax.dot_general(a_rows, a_rows, (((0,), (0,)), ((), ())),
                                   preferred_element_type=jnp.float32)
            reg = reg + jnp.sqrt(jnp.sum(gram * gram))
        o_ref[...] = _GAMMA + s1_ref[...] - s2_ref[...] + _ALPHA * reg


_score_call = pl.pallas_call(
    _score_body,
    grid=(3,),
    in_specs=[pl.BlockSpec((1, 3 * _BATCH, _D), lambda i: (i, 0, 0))],
    out_specs=pl.BlockSpec((_BATCH,), lambda i: (0,)),
    out_shape=jax.ShapeDtypeStruct((_BATCH,), jnp.float32),
    scratch_shapes=[pltpu.VMEM((_BATCH,), jnp.float32),
                    pltpu.VMEM((_BATCH,), jnp.float32)],
)


def kernel(head, tail, relation, n_head, n_tail, n_relation, reg_user,
           reg_item, reg_brand, entity_embed, relation_embed):
    idxs = [x.astype(jnp.int32) for x in (
        head, tail, relation, n_head, n_tail, n_relation,
        reg_user, reg_item, reg_brand)]
    gathered = _sc_gather_call()(*idxs, entity_embed, relation_embed)
    return _score_call(gathered.reshape(3, 3 * _BATCH, _D))


# speedup vs baseline: 4.5091x; 1.0183x over previous
import functools

import jax
import jax.numpy as jnp
from jax import lax
from jax.experimental import pallas as pl
from jax.experimental.pallas import tpu as pltpu
from jax.experimental.pallas import tpu_sc as plsc

_GAMMA = 1.0
_ALPHA = 0.0001
_BATCH = 4096
_D = 128
_NW = 32
_BPW = _BATCH // _NW
_NBUF = 4


@functools.cache
def _sc_gather_call(tables):
    n = len(tables)
    mesh = plsc.VectorSubcoreMesh(core_axis_name="c", subcore_axis_name="s")

    @functools.partial(
        pl.kernel,
        out_type=jax.ShapeDtypeStruct((n * _BATCH, _D), jnp.float32),
        mesh=mesh,
        scratch_types=(
            [pltpu.VMEM((n, _BPW), jnp.int32),
             pltpu.VMEM((_NBUF, _BPW, _D), jnp.float32),
             pltpu.SemaphoreType.DMA]
            + [pltpu.SemaphoreType.DMA] * (2 * _NBUF)
        ),
    )
    def _sc_gather(*refs):
        idx_refs = refs[:n]
        ent_hbm, rel_hbm, out_hbm, idx_v, rows_v, isem = refs[n:n + 6]
        bufsems = refs[n + 6:]
        gsem = bufsems[:_NBUF]
        wsem = bufsems[_NBUF:]
        wid = lax.axis_index("s") * 2 + lax.axis_index("c")
        base = wid * _BPW

        icopies = [
            pltpu.async_copy(idx_refs[a].at[pl.ds(base, _BPW)], idx_v.at[a],
                             isem)
            for a in range(n)
        ]
        for c in icopies:
            c.wait()

        def gather(a):
            table = rel_hbm if tables[a] else ent_hbm
            b = a % _NBUF
            return pltpu.async_copy(table.at[idx_v.at[a]], rows_v.at[b],
                                    gsem[b])

        g = {}
        w = {}
        for a in range(min(2, n)):
            g[a] = gather(a)
        for a in range(n):
            b = a % _NBUF
            g[a].wait()
            if a + 2 < n:
                if a + 2 - _NBUF >= 0:
                    w[a + 2 - _NBUF].wait()
                g[a + 2] = gather(a + 2)
            w[a] = pltpu.async_copy(
                rows_v.at[b], out_hbm.at[pl.ds(a * _BATCH + base, _BPW)],
                wsem[b])
        for a in range(max(0, n - _NBUF), n):
            w[a].wait()

    return _sc_gather


def _rows(g_ref, a):
    return g_ref[0, a * _BATCH:(a + 1) * _BATCH, :]

def _norm_rows(x):
    return x * lax.rsqrt(jnp.sum(x * x, axis=1, keepdims=True))


def _dist_body(g_ref, o_ref, s1_ref):
    i = pl.program_id(0)

    def half_dist():
        d = (_norm_rows(_rows(g_ref, 0)) + _norm_rows(_rows(g_ref, 2))
             - _norm_rows(_rows(g_ref, 1)))
        return jnp.sqrt(jnp.sum(d * d, axis=1))

    @pl.when(i == 0)
    def _():
        s1_ref[...] = half_dist()

    @pl.when(i == 1)
    def _():
        o_ref[...] = s1_ref[...] - half_dist()


_dist_call = pl.pallas_call(
    _dist_body,
    grid=(2,),
    in_specs=[pl.BlockSpec((1, 3 * _BATCH, _D), lambda i: (i, 0, 0))],
    out_specs=pl.BlockSpec((_BATCH,), lambda i: (0,)),
    out_shape=jax.ShapeDtypeStruct((_BATCH,), jnp.float32),
    scratch_shapes=[pltpu.VMEM((_BATCH,), jnp.float32)],
)


def _reg_body(g_ref, s_ref, o_ref):
    reg = jnp.float32(0.0)
    for a in range(3):
        a_rows = g_ref[a * _BATCH:(a + 1) * _BATCH, :]
        gram = lax.dot_general(a_rows, a_rows, (((0,), (0,)), ((), ())),
                               preferred_element_type=jnp.float32)
        reg = reg + jnp.sqrt(jnp.sum(gram * gram))
    o_ref[...] = _GAMMA + s_ref[...] + _ALPHA * reg


_reg_call = pl.pallas_call(
    _reg_body,
    out_shape=jax.ShapeDtypeStruct((_BATCH,), jnp.float32),
)


def kernel(head, tail, relation, n_head, n_tail, n_relation, reg_user,
           reg_item, reg_brand, entity_embed, relation_embed):
    score_idxs = [x.astype(jnp.int32) for x in (
        head, tail, relation, n_head, n_tail, n_relation)]
    reg_idxs = [x.astype(jnp.int32) for x in (reg_user, reg_item, reg_brand)]
    score_rows = _sc_gather_call((0, 0, 1, 0, 0, 1))(
        *score_idxs, entity_embed, relation_embed)
    reg_rows = _sc_gather_call((0, 0, 0))(
        *reg_idxs, entity_embed, relation_embed)
    sdiff = _dist_call(score_rows.reshape(2, 3 * _BATCH, _D))
    return _reg_call(reg_rows, sdiff)
